# baseline (device time: 16287 ns/iter reference)
import jax
import jax.numpy as jnp
from jax import lax
from jax.experimental import pallas as pl
from jax.experimental.pallas import tpu as pltpu

N_DEV = 4


def kernel(x, w_mat):
    k_dim, m_blk = x.shape
    _, n = w_mat.shape
    blk = k_dim // N_DEV

    def body(x_ref, w_hbm, out_ref, w_ref, comm_ref, send_sems, recv_sems,
             w_copy_sem):
        my = lax.axis_index("i")

        w_copy = pltpu.make_async_copy(w_hbm, w_ref, w_copy_sem)
        w_copy.start()

        barrier = pltpu.get_barrier_semaphore()
        for off in (1, 2, 3):
            pl.semaphore_signal(
                barrier, inc=1,
                device_id=((my + off) % N_DEV,),
                device_id_type=pl.DeviceIdType.MESH,
            )
        pl.semaphore_wait(barrier, N_DEV - 1)

        sends = []
        for off in (1, 2, 3):
            tgt = (my + off) % N_DEV
            slot = 3 - off
            rdma = pltpu.make_async_remote_copy(
                src_ref=x_ref.at[pl.ds(tgt * blk, blk), :],
                dst_ref=comm_ref.at[slot],
                send_sem=send_sems.at[off - 1],
                recv_sem=recv_sems.at[slot],
                device_id=(tgt,),
                device_id_type=pl.DeviceIdType.MESH,
            )
            rdma.start()
            sends.append(rdma)

        w_copy.wait()
        acc = jnp.dot(
            x_ref[pl.ds(my * blk, blk), :],
            w_ref[pl.ds(my * blk, blk), :],
            preferred_element_type=jnp.float32,
        )

        for slot in (0, 2, 1):
            src = (my + slot + 1) % N_DEV
            recv = pltpu.make_async_remote_copy(
                src_ref=x_ref.at[pl.ds(0, blk), :],
                dst_ref=comm_ref.at[slot],
                send_sem=send_sems.at[0],
                recv_sem=recv_sems.at[slot],
                device_id=(my,),
                device_id_type=pl.DeviceIdType.MESH,
            )
            recv.wait_recv()
            acc = acc + jnp.dot(
                comm_ref[slot],
                w_ref[pl.ds(src * blk, blk), :],
                preferred_element_type=jnp.float32,
            )

        for rdma in sends:
            rdma.wait_send()

        out_ref[...] = jnp.maximum(acc, 0.0)

    return pl.pallas_call(
        body,
        out_shape=jax.ShapeDtypeStruct((blk, n), jnp.float32),
        in_specs=[
            pl.BlockSpec(memory_space=pltpu.VMEM),
            pl.BlockSpec(memory_space=pltpu.MemorySpace.HBM),
        ],
        out_specs=pl.BlockSpec(memory_space=pltpu.VMEM),
        scratch_shapes=[
            pltpu.VMEM((k_dim, n), jnp.float32),
            pltpu.VMEM((N_DEV - 1, blk, m_blk), jnp.float32),
            pltpu.SemaphoreType.DMA((N_DEV - 1,)),
            pltpu.SemaphoreType.DMA((N_DEV - 1,)),
            pltpu.SemaphoreType.DMA,
        ],
        compiler_params=pltpu.CompilerParams(collective_id=0),
    )(x, w_mat)


# device time: 13362 ns/iter; 1.2189x vs baseline; 1.2189x over previous
import jax
import jax.numpy as jnp
from jax import lax
from jax.experimental import pallas as pl
from jax.experimental.pallas import tpu as pltpu

N_DEV = 4


def kernel(x, w_mat):
    k_dim, m_blk = x.shape
    _, n = w_mat.shape
    blk = k_dim // N_DEV

    def body(x_ref, w_ref, out_ref, xbf_ref, comm_ref, send_sems, recv_sems):
        my = lax.axis_index("i")

        xbf_ref[...] = x_ref[...].astype(jnp.bfloat16)

        barrier = pltpu.get_barrier_semaphore()
        for off in (1, 2, 3):
            pl.semaphore_signal(
                barrier, inc=1,
                device_id=((my + off) % N_DEV,),
                device_id_type=pl.DeviceIdType.MESH,
            )
        pl.semaphore_wait(barrier, N_DEV - 1)

        sends = []
        for off in (1, 2, 3):
            tgt = (my + off) % N_DEV
            slot = 3 - off
            rdma = pltpu.make_async_remote_copy(
                src_ref=xbf_ref.at[pl.ds(tgt * blk, blk), :],
                dst_ref=comm_ref.at[slot],
                send_sem=send_sems.at[off - 1],
                recv_sem=recv_sems.at[slot],
                device_id=(tgt,),
                device_id_type=pl.DeviceIdType.MESH,
            )
            rdma.start()
            sends.append(rdma)

        acc = jnp.dot(
            x_ref[pl.ds(my * blk, blk), :],
            w_ref[pl.ds(my * blk, blk), :],
            preferred_element_type=jnp.float32,
        )

        for slot in (0, 2, 1):
            src = (my + slot + 1) % N_DEV
            recv = pltpu.make_async_remote_copy(
                src_ref=xbf_ref.at[pl.ds(0, blk), :],
                dst_ref=comm_ref.at[slot],
                send_sem=send_sems.at[0],
                recv_sem=recv_sems.at[slot],
                device_id=(my,),
                device_id_type=pl.DeviceIdType.MESH,
            )
            recv.wait_recv()
            acc = acc + jnp.dot(
                comm_ref[slot],
                w_ref[pl.ds(src * blk, blk), :],
                preferred_element_type=jnp.float32,
            )

        for rdma in sends:
            rdma.wait_send()

        out_ref[...] = jnp.maximum(acc, 0.0)

    return pl.pallas_call(
        body,
        out_shape=jax.ShapeDtypeStruct((blk, n), jnp.float32),
        in_specs=[
            pl.BlockSpec(memory_space=pltpu.VMEM),
            pl.BlockSpec(memory_space=pltpu.VMEM),
        ],
        out_specs=pl.BlockSpec(memory_space=pltpu.VMEM),
        scratch_shapes=[
            pltpu.VMEM((k_dim, m_blk), jnp.bfloat16),
            pltpu.VMEM((N_DEV - 1, blk, m_blk), jnp.bfloat16),
            pltpu.SemaphoreType.DMA((N_DEV - 1,)),
            pltpu.SemaphoreType.DMA((N_DEV - 1,)),
        ],
        compiler_params=pltpu.CompilerParams(collective_id=0),
    )(x, w_mat)
